# baseline (speedup 1.0000x reference)
import functools

import jax
import jax.numpy as jnp
from jax import lax
from jax.experimental import pallas as pl
from jax.experimental.pallas import tpu as pltpu
from jax.experimental.pallas import tpu_sc as plsc

NC, NS, L = 2, 16, 16
NW = NC * NS
BT = 128


def _shift_down(a, sh):
    z = jnp.zeros((sh,) + a.shape[1:], a.dtype)
    return jnp.concatenate([z, a[:-sh]], axis=0)


def _shift_right(a, sh):
    z = jnp.zeros(a.shape[:1] + (sh,) + a.shape[2:], a.dtype)
    return jnp.concatenate([z, a[:, :-sh]], axis=1)


def _router_body(x_ref, rw_ref, meta_i_ref, meta_f_ref, *, T, E, NRT, NTE):
    x = x_ref[...]
    logits = lax.dot_general(x, rw_ref[...], (((1,), (1,)), ((), ())),
                             preferred_element_type=jnp.float32)
    C = logits.shape[1]
    cols = lax.broadcasted_iota(jnp.int32, (T, C), 1)
    is_e = cols < E
    el = jnp.where(is_e, logits, -1e30)
    m = jnp.max(el, axis=1, keepdims=True)
    p = jnp.where(is_e, jnp.exp(el - m), 0.0)
    p = p / jnp.sum(p, axis=1, keepdims=True)

    p1 = jnp.max(p, axis=1, keepdims=True)
    a1 = jnp.min(jnp.where(p == p1, cols, C), axis=1, keepdims=True)
    p_wo = jnp.where(cols == a1, -1.0, p)
    p2 = jnp.max(p_wo, axis=1, keepdims=True)
    a2 = jnp.min(jnp.where(p_wo == p2, cols, C), axis=1, keepdims=True)
    wsum = p1 + p2
    w1, w2 = p1 / wsum, p2 / wsum
    g = 1.0 / (1.0 + jnp.exp(-logits[:, E:E + 1]))

    m0 = (cols == a1)
    m1 = (cols == a2)
    mm = (m0 | m1).astype(jnp.int32)
    c = mm
    sh = 1
    while sh < T:
        c = c + _shift_down(c, sh)
        sh *= 2
    counts = c[T - 1:T, :]
    excl = c - mm
    rank0 = jnp.sum(jnp.where(m0, excl, 0), axis=1, keepdims=True)
    rank1 = jnp.sum(jnp.where(m1, excl, 0), axis=1, keepdims=True)

    nt = (counts + (BT - 1)) // BT
    ts = nt
    sh = 1
    while sh < C:
        ts = ts + _shift_right(ts, sh)
        sh *= 2
    tile_start = ts - nt
    off = tile_start * BT
    pos0 = jnp.sum(jnp.where(m0, off, 0), axis=1, keepdims=True) + rank0
    pos1 = jnp.sum(jnp.where(m1, off, 0), axis=1, keepdims=True) + rank1

    ti = lax.broadcasted_iota(jnp.int32, (NTE, C), 0)
    tcols = lax.broadcasted_iota(jnp.int32, (NTE, C), 1)
    ts_b = jnp.broadcast_to(tile_start, (NTE, C))
    nt_b = jnp.broadcast_to(nt, (NTE, C))
    ind = ((ti >= ts_b) & (ti < ts_b + nt_b) & (tcols < E)).astype(jnp.int32)
    any_ind = jnp.sum(ind, axis=1, keepdims=True)
    te = jnp.sum(ind * tcols, axis=1, keepdims=True) - (1 - any_ind)
    te = jnp.where(ti[:, :1] >= NRT, E, te)

    meta_i_ref[:, 0:1] = pos0
    meta_i_ref[:, 1:2] = pos1
    meta_i_ref[0:NTE, 2:3] = te
    meta_f_ref[:, 0:L] = jnp.broadcast_to(w1, (T, L))
    meta_f_ref[:, L:2 * L] = jnp.broadcast_to(w2, (T, L))
    meta_f_ref[:, 2 * L:3 * L] = jnp.broadcast_to(g, (T, L))


def _dispatch_body(x_hbm, pos0_hbm, pos1_hbm, xs_hbm, buf, idx0, idx1, sem0, sem1,
                   *, chunk):
    wid = lax.axis_index("s") * NC + lax.axis_index("c")
    base = pl.multiple_of(wid * chunk, 8)
    pltpu.sync_copy(x_hbm.at[pl.ds(base, chunk)], buf)
    pltpu.sync_copy(pos0_hbm.at[pl.ds(base, chunk)], idx0)
    pltpu.sync_copy(pos1_hbm.at[pl.ds(base, chunk)], idx1)
    c0 = pltpu.async_copy(buf, xs_hbm.at[idx0], sem0)
    c1 = pltpu.async_copy(buf, xs_hbm.at[idx1], sem1)
    c0.wait()
    c1.wait()


def _ffn_body(te_ref, xs_ref, x_ref, wgu_ref, wd_ref, o_ref, *, NRT, I):
    i = pl.program_id(0)
    tev = te_ref[i]
    rows = jnp.where(i < NRT, xs_ref[...], x_ref[...])

    @pl.when(tev >= 0)
    def _():
        gu = lax.dot_general(rows, wgu_ref[0], (((1,), (1,)), ((), ())),
                             preferred_element_type=jnp.float32)
        gt = gu[:, :I]
        up = gu[:, I:]
        act = gt * (1.0 / (1.0 + jnp.exp(-gt))) * up
        o_ref[...] = lax.dot_general(act, wd_ref[0], (((1,), (1,)), ((), ())),
                                     preferred_element_type=jnp.float32)


def _combine_body(oall_hbm, pos0_hbm, pos1_hbm, mf_hbm, out_hbm,
                  a_buf, b_buf, s_buf, w_buf, idx0, idx1, sem0, sem1,
                  *, rows, n_rounds, shared_base, H):
    wid = lax.axis_index("s") * NC + lax.axis_index("c")
    nch = H // L

    def round_body(rnd, _):
        base = pl.multiple_of(wid * (rows * n_rounds) + rnd * rows, 8)
        pltpu.sync_copy(pos0_hbm.at[pl.ds(base, rows)], idx0)
        pltpu.sync_copy(pos1_hbm.at[pl.ds(base, rows)], idx1)
        c0 = pltpu.async_copy(oall_hbm.at[idx0], a_buf, sem0)
        c1 = pltpu.async_copy(oall_hbm.at[idx1], b_buf, sem1)
        pltpu.sync_copy(oall_hbm.at[pl.ds(shared_base + base, rows)], s_buf)
        pltpu.sync_copy(mf_hbm.at[pl.ds(base, rows)], w_buf)
        c0.wait()
        c1.wait()

        def row_body(r, _):
            wv0 = w_buf[r, pl.ds(0, L)]
            wv1 = w_buf[r, pl.ds(L, L)]
            wvg = w_buf[r, pl.ds(2 * L, L)]

            def chunk_body(ci, _):
                o = pl.ds(pl.multiple_of(ci * L, L), L)
                a_buf[r, o] = (a_buf[r, o] * wv0 + b_buf[r, o] * wv1
                               + s_buf[r, o] * wvg)
                return 0

            lax.fori_loop(0, nch, chunk_body, 0, unroll=4)
            return 0

        lax.fori_loop(0, rows, row_body, 0)
        pltpu.sync_copy(a_buf, out_hbm.at[pl.ds(base, rows)])
        return 0

    lax.fori_loop(0, n_rounds, round_body, 0)


def kernel(hidden_states, gate_w, expert_gate_up_w, expert_down_w,
           shared_gate_up_w, shared_down_w, shared_expert_gate_w):
    T, H = hidden_states.shape
    E = gate_w.shape[0]
    I2 = expert_gate_up_w.shape[1]
    I = I2 // 2
    K = 2
    NRT = (T * K) // BT + E
    NRS = NRT * BT
    NST = T // BT
    N_TILES = NRT + NST
    NTE = ((N_TILES + 7) // 8) * 8
    x = hidden_states.reshape(T, H)

    rw = jnp.concatenate([gate_w, shared_expert_gate_w,
                          jnp.zeros((2 * L - E - 1, H), jnp.float32)], axis=0)
    meta_i, meta_f = pl.pallas_call(
        functools.partial(_router_body, T=T, E=E, NRT=NRT, NTE=NTE),
        out_shape=(jax.ShapeDtypeStruct((T, 8), jnp.int32),
                   jax.ShapeDtypeStruct((T, 3 * L), jnp.float32)),
    )(x, rw)
    pos0 = meta_i[:, 0]
    pos1 = meta_i[:, 1]
    te = meta_i[:NTE, 2]

    chunk = T // NW
    x_sorted = pl.kernel(
        functools.partial(_dispatch_body, chunk=chunk),
        out_type=jax.ShapeDtypeStruct((NRS, H), jnp.float32),
        mesh=plsc.VectorSubcoreMesh(core_axis_name="c", subcore_axis_name="s"),
        scratch_types=[
            pltpu.VMEM((chunk, H), jnp.float32),
            pltpu.VMEM((chunk,), jnp.int32),
            pltpu.VMEM((chunk,), jnp.int32),
            pltpu.SemaphoreType.DMA,
            pltpu.SemaphoreType.DMA,
        ],
    )(x, pos0, pos1)

    wgu = jnp.concatenate([expert_gate_up_w, shared_gate_up_w[None]], axis=0)
    wd = jnp.concatenate([expert_down_w, shared_down_w[None]], axis=0)
    out_all = pl.pallas_call(
        functools.partial(_ffn_body, NRT=NRT, I=I),
        grid_spec=pltpu.PrefetchScalarGridSpec(
            num_scalar_prefetch=1,
            grid=(N_TILES,),
            in_specs=[
                pl.BlockSpec((BT, H), lambda i, s: (jnp.minimum(i, NRT - 1), 0)),
                pl.BlockSpec((BT, H), lambda i, s: (jnp.maximum(i - NRT, 0), 0)),
                pl.BlockSpec((1, I2, H),
                             lambda i, s: (jnp.where(s[i] < 0, E, s[i]), 0, 0)),
                pl.BlockSpec((1, H, I),
                             lambda i, s: (jnp.where(s[i] < 0, E, s[i]), 0, 0)),
            ],
            out_specs=pl.BlockSpec((BT, H), lambda i, s: (i, 0)),
        ),
        out_shape=jax.ShapeDtypeStruct((N_TILES * BT, H), jnp.float32),
    )(te, x_sorted, x, wgu, wd)

    rows = 32
    n_rounds = chunk // rows
    final = pl.kernel(
        functools.partial(_combine_body, rows=rows, n_rounds=n_rounds,
                          shared_base=NRS, H=H),
        out_type=jax.ShapeDtypeStruct((T, H), jnp.float32),
        mesh=plsc.VectorSubcoreMesh(core_axis_name="c", subcore_axis_name="s"),
        scratch_types=[
            pltpu.VMEM((rows, H), jnp.float32),
            pltpu.VMEM((rows, H), jnp.float32),
            pltpu.VMEM((rows, H), jnp.float32),
            pltpu.VMEM((rows, 3 * L), jnp.float32),
            pltpu.VMEM((rows,), jnp.int32),
            pltpu.VMEM((rows,), jnp.int32),
            pltpu.SemaphoreType.DMA,
            pltpu.SemaphoreType.DMA,
        ],
    )(out_all, pos0, pos1, meta_f)

    return final.reshape(hidden_states.shape)


# speedup vs baseline: 1.1155x; 1.1155x over previous
import functools

import jax
import jax.numpy as jnp
from jax import lax
from jax.experimental import pallas as pl
from jax.experimental.pallas import tpu as pltpu
from jax.experimental.pallas import tpu_sc as plsc

NC, NS, L = 2, 16, 16
NW = NC * NS
BT = 128


def _shift_down(a, sh):
    z = jnp.zeros((sh,) + a.shape[1:], a.dtype)
    return jnp.concatenate([z, a[:-sh]], axis=0)


def _shift_right(a, sh):
    z = jnp.zeros(a.shape[:1] + (sh,) + a.shape[2:], a.dtype)
    return jnp.concatenate([z, a[:, :-sh]], axis=1)


def _router_body(x_ref, rw_ref, meta_i_ref, meta_f_ref, *, T, E, NRT, NTE):
    x = x_ref[...]
    logits = lax.dot_general(x, rw_ref[...], (((1,), (1,)), ((), ())),
                             preferred_element_type=jnp.float32)
    C = logits.shape[1]
    cols = lax.broadcasted_iota(jnp.int32, (T, C), 1)
    is_e = cols < E
    el = jnp.where(is_e, logits, -1e30)
    m = jnp.max(el, axis=1, keepdims=True)
    p = jnp.where(is_e, jnp.exp(el - m), 0.0)
    p = p / jnp.sum(p, axis=1, keepdims=True)

    p1 = jnp.max(p, axis=1, keepdims=True)
    a1 = jnp.min(jnp.where(p == p1, cols, C), axis=1, keepdims=True)
    p_wo = jnp.where(cols == a1, -1.0, p)
    p2 = jnp.max(p_wo, axis=1, keepdims=True)
    a2 = jnp.min(jnp.where(p_wo == p2, cols, C), axis=1, keepdims=True)
    wsum = p1 + p2
    w1, w2 = p1 / wsum, p2 / wsum
    g = 1.0 / (1.0 + jnp.exp(-logits[:, E:E + 1]))

    m0 = (cols == a1)
    m1 = (cols == a2)
    mm = (m0 | m1).astype(jnp.int32)
    c = mm
    sh = 1
    while sh < T:
        c = c + _shift_down(c, sh)
        sh *= 2
    counts = c[T - 1:T, :]
    excl = c - mm
    rank0 = jnp.sum(jnp.where(m0, excl, 0), axis=1, keepdims=True)
    rank1 = jnp.sum(jnp.where(m1, excl, 0), axis=1, keepdims=True)

    nt = (counts + (BT - 1)) // BT
    ts = nt
    sh = 1
    while sh < C:
        ts = ts + _shift_right(ts, sh)
        sh *= 2
    tile_start = ts - nt
    off = tile_start * BT
    pos0 = jnp.sum(jnp.where(m0, off, 0), axis=1, keepdims=True) + rank0
    pos1 = jnp.sum(jnp.where(m1, off, 0), axis=1, keepdims=True) + rank1

    ti = lax.broadcasted_iota(jnp.int32, (NTE, C), 0)
    tcols = lax.broadcasted_iota(jnp.int32, (NTE, C), 1)
    ts_b = jnp.broadcast_to(tile_start, (NTE, C))
    nt_b = jnp.broadcast_to(nt, (NTE, C))
    ind = ((ti >= ts_b) & (ti < ts_b + nt_b) & (tcols < E)).astype(jnp.int32)
    any_ind = jnp.sum(ind, axis=1, keepdims=True)
    te = jnp.sum(ind * tcols, axis=1, keepdims=True) - (1 - any_ind)
    te = jnp.where(ti[:, :1] >= NRT, E, te)

    meta_i_ref[:, 0:1] = pos0
    meta_i_ref[:, 1:2] = pos1
    meta_i_ref[0:NTE, 2:3] = te
    meta_f_ref[:, 0:L] = jnp.broadcast_to(w1, (T, L))
    meta_f_ref[:, L:2 * L] = jnp.broadcast_to(w2, (T, L))
    meta_f_ref[:, 2 * L:3 * L] = jnp.broadcast_to(g, (T, L))


def _dispatch_body(x_hbm, pos0_hbm, pos1_hbm, xs_hbm, buf, idx0, idx1, sem0, sem1,
                   *, chunk):
    wid = lax.axis_index("s") * NC + lax.axis_index("c")
    base = pl.multiple_of(wid * chunk, 8)
    pltpu.sync_copy(x_hbm.at[pl.ds(base, chunk)], buf)
    pltpu.sync_copy(pos0_hbm.at[pl.ds(base, chunk)], idx0)
    pltpu.sync_copy(pos1_hbm.at[pl.ds(base, chunk)], idx1)
    c0 = pltpu.async_copy(buf, xs_hbm.at[idx0], sem0)
    c1 = pltpu.async_copy(buf, xs_hbm.at[idx1], sem1)
    c0.wait()
    c1.wait()


def _ffn_body(te_ref, xs_ref, x_ref, wgu_ref, wd_ref, o_ref, *, NRT, I):
    i = pl.program_id(0)
    tev = te_ref[i]
    rows = jnp.where(i < NRT, xs_ref[...], x_ref[...])

    @pl.when(tev >= 0)
    def _():
        gu = lax.dot_general(rows, wgu_ref[0], (((1,), (1,)), ((), ())),
                             preferred_element_type=jnp.float32)
        gt = gu[:, :I]
        up = gu[:, I:]
        act = gt * (1.0 / (1.0 + jnp.exp(-gt))) * up
        o_ref[...] = lax.dot_general(act, wd_ref[0], (((1,), (1,)), ((), ())),
                                     preferred_element_type=jnp.float32)


def _combine_body(oall_hbm, pos0_hbm, pos1_hbm, mf_hbm, out_hbm,
                  a_buf, b_buf, s_buf, w_buf, idx0, idx1, sem0, sem1,
                  *, rows, n_rounds, shared_base, H):
    wid = lax.axis_index("s") * NC + lax.axis_index("c")
    nch = H // L

    def round_body(rnd, _):
        base = pl.multiple_of(wid * (rows * n_rounds) + rnd * rows, 8)
        pltpu.sync_copy(pos0_hbm.at[pl.ds(base, rows)], idx0)
        pltpu.sync_copy(pos1_hbm.at[pl.ds(base, rows)], idx1)
        c0 = pltpu.async_copy(oall_hbm.at[idx0], a_buf, sem0)
        c1 = pltpu.async_copy(oall_hbm.at[idx1], b_buf, sem1)
        pltpu.sync_copy(oall_hbm.at[pl.ds(shared_base + base, rows)], s_buf)
        pltpu.sync_copy(mf_hbm.at[pl.ds(base, rows)], w_buf)
        c0.wait()
        c1.wait()

        @plsc.parallel_loop(0, rows)
        def row_body(r):
            wv0 = w_buf[r, pl.ds(0, L)]
            wv1 = w_buf[r, pl.ds(L, L)]
            wvg = w_buf[r, pl.ds(2 * L, L)]

            @plsc.parallel_loop(0, nch, unroll=8)
            def chunk_body(ci):
                o = pl.ds(pl.multiple_of(ci * L, L), L)
                a_buf[r, o] = (a_buf[r, o] * wv0 + b_buf[r, o] * wv1
                               + s_buf[r, o] * wvg)
        pltpu.sync_copy(a_buf, out_hbm.at[pl.ds(base, rows)])
        return 0

    lax.fori_loop(0, n_rounds, round_body, 0)


def kernel(hidden_states, gate_w, expert_gate_up_w, expert_down_w,
           shared_gate_up_w, shared_down_w, shared_expert_gate_w):
    T, H = hidden_states.shape
    E = gate_w.shape[0]
    I2 = expert_gate_up_w.shape[1]
    I = I2 // 2
    K = 2
    NRT = (T * K) // BT + E
    NRS = NRT * BT
    NST = T // BT
    N_TILES = NRT + NST
    NTE = ((N_TILES + 7) // 8) * 8
    x = hidden_states.reshape(T, H)

    rw = jnp.concatenate([gate_w, shared_expert_gate_w,
                          jnp.zeros((2 * L - E - 1, H), jnp.float32)], axis=0)
    meta_i, meta_f = pl.pallas_call(
        functools.partial(_router_body, T=T, E=E, NRT=NRT, NTE=NTE),
        out_shape=(jax.ShapeDtypeStruct((T, 8), jnp.int32),
                   jax.ShapeDtypeStruct((T, 3 * L), jnp.float32)),
    )(x, rw)
    pos0 = meta_i[:, 0]
    pos1 = meta_i[:, 1]
    te = meta_i[:NTE, 2]

    chunk = T // NW
    x_sorted = pl.kernel(
        functools.partial(_dispatch_body, chunk=chunk),
        out_type=jax.ShapeDtypeStruct((NRS, H), jnp.float32),
        mesh=plsc.VectorSubcoreMesh(core_axis_name="c", subcore_axis_name="s"),
        scratch_types=[
            pltpu.VMEM((chunk, H), jnp.float32),
            pltpu.VMEM((chunk,), jnp.int32),
            pltpu.VMEM((chunk,), jnp.int32),
            pltpu.SemaphoreType.DMA,
            pltpu.SemaphoreType.DMA,
        ],
    )(x, pos0, pos1)

    wgu = jnp.concatenate([expert_gate_up_w, shared_gate_up_w[None]], axis=0)
    wd = jnp.concatenate([expert_down_w, shared_down_w[None]], axis=0)
    out_all = pl.pallas_call(
        functools.partial(_ffn_body, NRT=NRT, I=I),
        grid_spec=pltpu.PrefetchScalarGridSpec(
            num_scalar_prefetch=1,
            grid=(N_TILES,),
            in_specs=[
                pl.BlockSpec((BT, H), lambda i, s: (jnp.minimum(i, NRT - 1), 0)),
                pl.BlockSpec((BT, H), lambda i, s: (jnp.maximum(i - NRT, 0), 0)),
                pl.BlockSpec((1, I2, H),
                             lambda i, s: (jnp.where(s[i] < 0, E, s[i]), 0, 0)),
                pl.BlockSpec((1, H, I),
                             lambda i, s: (jnp.where(s[i] < 0, E, s[i]), 0, 0)),
            ],
            out_specs=pl.BlockSpec((BT, H), lambda i, s: (i, 0)),
        ),
        out_shape=jax.ShapeDtypeStruct((N_TILES * BT, H), jnp.float32),
    )(te, x_sorted, x, wgu, wd)

    rows = 32
    n_rounds = chunk // rows
    final = pl.kernel(
        functools.partial(_combine_body, rows=rows, n_rounds=n_rounds,
                          shared_base=NRS, H=H),
        out_type=jax.ShapeDtypeStruct((T, H), jnp.float32),
        mesh=plsc.VectorSubcoreMesh(core_axis_name="c", subcore_axis_name="s"),
        scratch_types=[
            pltpu.VMEM((rows, H), jnp.float32),
            pltpu.VMEM((rows, H), jnp.float32),
            pltpu.VMEM((rows, H), jnp.float32),
            pltpu.VMEM((rows, 3 * L), jnp.float32),
            pltpu.VMEM((rows,), jnp.int32),
            pltpu.VMEM((rows,), jnp.int32),
            pltpu.SemaphoreType.DMA,
            pltpu.SemaphoreType.DMA,
        ],
    )(out_all, pos0, pos1, meta_f)

    return final.reshape(hidden_states.shape)
